# baseline (speedup 1.0000x reference)
import functools

import jax
import jax.numpy as jnp
from jax import lax
from jax.experimental import pallas as pl
from jax.experimental.pallas import tpu as pltpu
from jax.experimental.pallas import tpu_sc as plsc

N = 10000
E = 320000
D = 128

NC = 2
NS = 16
NW = NC * NS

EPT = E // NW
C = 80
CHUNKS = EPT // C
NP = 10240
RPT = NP // NS

def _deg_body(idx_hbm, ones_hbm, zeros_hbm,
                odeg_hbm, ideg_hbm,
                idx_v, ones_v, odeg_s, ideg_s, psem, ssem):
    cid = lax.axis_index("c")
    sid = lax.axis_index("s")
    wid = cid * NS + sid

    cps = [
        pltpu.async_copy(idx_hbm.at[wid], idx_v, psem),
        pltpu.async_copy(ones_hbm, ones_v, psem),
        pltpu.async_copy(zeros_hbm.at[pl.ds(sid * RPT, RPT)],
                         odeg_s.at[pl.ds(sid * RPT, RPT)], psem),
        pltpu.async_copy(zeros_hbm.at[pl.ds(sid * RPT, RPT)],
                         ideg_s.at[pl.ds(sid * RPT, RPT)], psem),
    ]
    for cp in cps:
        cp.wait()
    plsc.subcore_barrier()

    @pl.loop(0, CHUNKS, step=5)
    def step(p):
        fired = []
        for q in range(5):
            fired.append(pltpu.async_copy(
                ones_v, odeg_s.at[idx_v.at[p + q, 0]], ssem, add=True))
            fired.append(pltpu.async_copy(
                ones_v, ideg_s.at[idx_v.at[p + q, 1]], ssem, add=True))
        for cp in fired:
            cp.wait()

    plsc.subcore_barrier()

    pltpu.sync_copy(odeg_s.at[pl.ds(sid * RPT, RPT)],
                    odeg_hbm.at[cid, pl.ds(sid * RPT, RPT)])
    pltpu.sync_copy(ideg_s.at[pl.ds(sid * RPT, RPT)],
                    ideg_hbm.at[cid, pl.ds(sid * RPT, RPT)])


def _agg_body(hw_hbm, idx_hbm, zeros_hbm, out_hbm,
                ib0, ib1, ib2, ib3, rb0, rb1, rb2, rb3, acc_s,
                psem, is0, is1, is2, is3, gs0, gs1, gs2, gs3):
    cid = lax.axis_index("c")
    sid = lax.axis_index("s")
    wid = cid * NS + sid

    ibufs = [ib0, ib1, ib2, ib3]
    rbufs = [rb0, rb1, rb2, rb3]
    isems = [is0, is1, is2, is3]
    gsems = [gs0, gs1, gs2, gs3]

    def stage_idx(chunk, q):
        pltpu.async_copy(idx_hbm.at[wid, chunk], ibufs[q], isems[q])

    def wait_idx(chunk, q):
        pltpu.make_async_copy(idx_hbm.at[wid, chunk], ibufs[q],
                              isems[q]).wait()

    def fire_gather(q):
        pltpu.async_copy(hw_hbm.at[ibufs[q].at[0]], rbufs[q], gsems[q])

    def wait_gather(q):
        pltpu.make_async_copy(hw_hbm.at[ibufs[q].at[0]], rbufs[q],
                              gsems[q]).wait()

    def scatter(q):
        pltpu.sync_copy(rbufs[q], acc_s.at[ibufs[q].at[1]], add=True)

    zcp = pltpu.async_copy(zeros_hbm.at[pl.ds(sid * RPT, RPT)],
                           acc_s.at[pl.ds(sid * RPT, RPT)], psem)
    for q in range(3):
        stage_idx(q, q)
    for q in range(3):
        wait_idx(q, q)
        fire_gather(q)
    stage_idx(3, 3)
    zcp.wait()
    plsc.subcore_barrier()

    @pl.loop(0, CHUNKS - 5, step=4)
    def quad(p):
        for q in range(4):
            nq = (q + 3) % 4
            wait_idx(p + q + 3, nq)
            fire_gather(nq)
            wait_gather(q)
            scatter(q)
            stage_idx(p + q + 4, q)

    base = CHUNKS - 5
    wait_idx(base + 3, 3)
    fire_gather(3)
    wait_gather(0)
    scatter(0)
    stage_idx(base + 4, 0)
    wait_gather(1)
    scatter(1)
    wait_gather(2)
    scatter(2)
    wait_idx(base + 4, 0)
    fire_gather(0)
    wait_gather(3)
    scatter(3)
    wait_gather(0)
    scatter(0)

    plsc.subcore_barrier()

    pltpu.sync_copy(acc_s.at[pl.ds(sid * RPT, RPT)],
                    out_hbm.at[cid, pl.ds(sid * RPT, RPT)])


@functools.cache
def _sc_kernels():
    mesh = plsc.VectorSubcoreMesh(core_axis_name="c", subcore_axis_name="s",
                                  num_cores=NC, num_subcores=NS)
    deg = pl.kernel(
        _deg_body,
        out_type=[
            jax.ShapeDtypeStruct((NC, NP, 16), jnp.float32),
            jax.ShapeDtypeStruct((NC, NP, 16), jnp.float32),
        ],
        mesh=mesh,
        scratch_types=[
            pltpu.VMEM((CHUNKS, 2, C), jnp.int32),
            pltpu.VMEM((C, 16), jnp.float32),
            pltpu.VMEM_SHARED((NP, 16), jnp.float32),
            pltpu.VMEM_SHARED((NP, 16), jnp.float32),
            pltpu.SemaphoreType.DMA,
            pltpu.SemaphoreType.DMA,
        ],
        compiler_params=pltpu.CompilerParams(use_tc_tiling_on_sc=False),
    )
    agg = pl.kernel(
        _agg_body,
        out_type=jax.ShapeDtypeStruct((NC, NP, D), jnp.float32),
        mesh=mesh,
        scratch_types=(
            [pltpu.VMEM((2, C), jnp.int32)] * 4
            + [pltpu.VMEM((C, D), jnp.float32)] * 4
            + [pltpu.VMEM_SHARED((NP, D), jnp.float32)]
            + [pltpu.SemaphoreType.DMA] * 9
        ),
        compiler_params=pltpu.CompilerParams(use_tc_tiling_on_sc=False),
    )
    return deg, agg


_TC_BLK = 1000


def _hw_body(h_ref, w_ref, od_ref, o_ref):
    deg = od_ref[0] + od_ref[1]
    norm = lax.rsqrt(jnp.clip(deg[:, 0:1], 1.0, None))
    o_ref[...] = jnp.dot(h_ref[...] * norm, w_ref[...],
                         preferred_element_type=jnp.float32)


def _final_body(p_ref, id_ref, b_ref, o_ref):
    agg = p_ref[0] + p_ref[1]
    deg = id_ref[0] + id_ref[1]
    norm = lax.rsqrt(jnp.clip(deg[:, 0:1], 1.0, None))
    x = agg * norm + b_ref[...]
    o_ref[...] = jnp.where(x >= 0, x, 0.01 * x)


_hw_call = pl.pallas_call(
    _hw_body,
    grid=(N // _TC_BLK,),
    in_specs=[
        pl.BlockSpec((_TC_BLK, D), lambda i: (i, 0)),
        pl.BlockSpec((D, D), lambda i: (0, 0)),
        pl.BlockSpec((NC, _TC_BLK, 16), lambda i: (0, i, 0)),
    ],
    out_specs=pl.BlockSpec((_TC_BLK, D), lambda i: (i, 0)),
    out_shape=jax.ShapeDtypeStruct((N, D), jnp.float32),
)

_final_call = pl.pallas_call(
    _final_body,
    grid=(N // _TC_BLK,),
    in_specs=[
        pl.BlockSpec((NC, _TC_BLK, D), lambda i: (0, i, 0)),
        pl.BlockSpec((NC, _TC_BLK, 16), lambda i: (0, i, 0)),
        pl.BlockSpec((1, D), lambda i: (0, 0)),
    ],
    out_specs=pl.BlockSpec((_TC_BLK, D), lambda i: (i, 0)),
    out_shape=jax.ShapeDtypeStruct((N, D), jnp.float32),
)


def kernel(h, edge_index, W, b):
    idx4 = jnp.stack([edge_index[0].reshape(NW, CHUNKS, C),
                      edge_index[1].reshape(NW, CHUNKS, C)], axis=2)
    ones16 = jnp.ones((C, 16), jnp.float32)
    zeros16 = jnp.zeros((NP, 16), jnp.float32)
    zerosD = jnp.zeros((NP, D), jnp.float32)

    deg_kernel, agg_kernel = _sc_kernels()
    odeg_p, ideg_p = deg_kernel(idx4, ones16, zeros16)
    hw = _hw_call(h, W, odeg_p)
    partials = agg_kernel(hw, idx4, zerosD)
    return _final_call(partials, ideg_p, b.reshape(1, D))


# speedup vs baseline: 1.1099x; 1.1099x over previous
import functools

import jax
import jax.numpy as jnp
from jax import lax
from jax.experimental import pallas as pl
from jax.experimental.pallas import tpu as pltpu
from jax.experimental.pallas import tpu_sc as plsc

N = 10000
E = 320000
D = 128

NC = 2
NS = 16
NW = NC * NS

EPT = E // NW
C = 125
CHUNKS = EPT // C
NP = 10240
RPT = NP // NS

def _deg_body(idx_hbm, ones_hbm, zeros_hbm,
                odeg_hbm, ideg_hbm,
                idx_v, ones_v, odeg_s, ideg_s, psem, ssem):
    cid = lax.axis_index("c")
    sid = lax.axis_index("s")
    wid = cid * NS + sid

    cps = [
        pltpu.async_copy(idx_hbm.at[wid], idx_v, psem),
        pltpu.async_copy(ones_hbm, ones_v, psem),
        pltpu.async_copy(zeros_hbm.at[pl.ds(sid * RPT, RPT)],
                         odeg_s.at[pl.ds(sid * RPT, RPT)], psem),
        pltpu.async_copy(zeros_hbm.at[pl.ds(sid * RPT, RPT)],
                         ideg_s.at[pl.ds(sid * RPT, RPT)], psem),
    ]
    for cp in cps:
        cp.wait()
    plsc.subcore_barrier()

    @pl.loop(0, CHUNKS, step=4)
    def step(p):
        fired = []
        for q in range(4):
            fired.append(pltpu.async_copy(
                ones_v, odeg_s.at[idx_v.at[2 * (p + q)]], ssem, add=True))
            fired.append(pltpu.async_copy(
                ones_v, ideg_s.at[idx_v.at[2 * (p + q) + 1]], ssem, add=True))
        for cp in fired:
            cp.wait()

    plsc.subcore_barrier()

    pltpu.sync_copy(odeg_s.at[pl.ds(sid * RPT, RPT)],
                    odeg_hbm.at[cid, pl.ds(sid * RPT, RPT)])
    pltpu.sync_copy(ideg_s.at[pl.ds(sid * RPT, RPT)],
                    ideg_hbm.at[cid, pl.ds(sid * RPT, RPT)])


def _agg_body(hw_hbm, idx_hbm, zeros_hbm, out_hbm,
                idx_a, idx_b, rows_a, rows_b, acc_s,
                psem, isem_a, isem_b, gsem_a, gsem_b):
    cid = lax.axis_index("c")
    sid = lax.axis_index("s")
    wid = cid * NS + sid

    zcp = pltpu.async_copy(zeros_hbm.at[pl.ds(sid * RPT, RPT)],
                           acc_s.at[pl.ds(sid * RPT, RPT)], psem)
    pltpu.sync_copy(idx_hbm.at[wid, pl.ds(0, 2)], idx_a)
    pltpu.async_copy(hw_hbm.at[idx_a.at[0]], rows_a, gsem_a)
    pltpu.async_copy(idx_hbm.at[wid, pl.ds(2, 2)], idx_b, isem_b)
    zcp.wait()
    plsc.subcore_barrier()

    @pl.loop(0, CHUNKS - 2, step=2)
    def pair(p):
        pltpu.make_async_copy(hw_hbm.at[idx_a.at[0]], rows_a, gsem_a).wait()
        pltpu.make_async_copy(idx_hbm.at[wid, pl.ds(2 * (p + 1), 2)],
                              idx_b, isem_b).wait()
        pltpu.async_copy(hw_hbm.at[idx_b.at[0]], rows_b, gsem_b)
        pltpu.sync_copy(rows_a, acc_s.at[idx_a.at[1]], add=True)
        pltpu.async_copy(idx_hbm.at[wid, pl.ds(2 * (p + 2), 2)], idx_a,
                         isem_a)
        pltpu.make_async_copy(hw_hbm.at[idx_b.at[0]], rows_b, gsem_b).wait()
        pltpu.make_async_copy(idx_hbm.at[wid, pl.ds(2 * (p + 2), 2)],
                              idx_a, isem_a).wait()
        pltpu.async_copy(hw_hbm.at[idx_a.at[0]], rows_a, gsem_a)
        pltpu.sync_copy(rows_b, acc_s.at[idx_b.at[1]], add=True)
        pltpu.async_copy(idx_hbm.at[wid, pl.ds(2 * (p + 3), 2)], idx_b,
                         isem_b)

    p = CHUNKS - 2
    pltpu.make_async_copy(hw_hbm.at[idx_a.at[0]], rows_a, gsem_a).wait()
    pltpu.make_async_copy(idx_hbm.at[wid, pl.ds(2 * (p + 1), 2)],
                              idx_b, isem_b).wait()
    pltpu.async_copy(hw_hbm.at[idx_b.at[0]], rows_b, gsem_b)
    pltpu.sync_copy(rows_a, acc_s.at[idx_a.at[1]], add=True)
    pltpu.make_async_copy(hw_hbm.at[idx_b.at[0]], rows_b, gsem_b).wait()
    pltpu.sync_copy(rows_b, acc_s.at[idx_b.at[1]], add=True)

    plsc.subcore_barrier()

    pltpu.sync_copy(acc_s.at[pl.ds(sid * RPT, RPT)],
                    out_hbm.at[cid, pl.ds(sid * RPT, RPT)])


@functools.cache
def _sc_kernels():
    mesh = plsc.VectorSubcoreMesh(core_axis_name="c", subcore_axis_name="s",
                                  num_cores=NC, num_subcores=NS)
    deg = pl.kernel(
        _deg_body,
        out_type=[
            jax.ShapeDtypeStruct((NC, NP, 16), jnp.float32),
            jax.ShapeDtypeStruct((NC, NP, 16), jnp.float32),
        ],
        mesh=mesh,
        scratch_types=[
            pltpu.VMEM((2 * CHUNKS, C), jnp.int32),
            pltpu.VMEM((C, 16), jnp.float32),
            pltpu.VMEM_SHARED((NP, 16), jnp.float32),
            pltpu.VMEM_SHARED((NP, 16), jnp.float32),
            pltpu.SemaphoreType.DMA,
            pltpu.SemaphoreType.DMA,
        ],
        compiler_params=pltpu.CompilerParams(use_tc_tiling_on_sc=False),
    )
    agg = pl.kernel(
        _agg_body,
        out_type=jax.ShapeDtypeStruct((NC, NP, D), jnp.float32),
        mesh=mesh,
        scratch_types=[
            pltpu.VMEM((2, C), jnp.int32),
            pltpu.VMEM((2, C), jnp.int32),
            pltpu.VMEM((C, D), jnp.float32),
            pltpu.VMEM((C, D), jnp.float32),
            pltpu.VMEM_SHARED((NP, D), jnp.float32),
            pltpu.SemaphoreType.DMA,
            pltpu.SemaphoreType.DMA,
            pltpu.SemaphoreType.DMA,
            pltpu.SemaphoreType.DMA,
            pltpu.SemaphoreType.DMA,
        ],
        compiler_params=pltpu.CompilerParams(use_tc_tiling_on_sc=False),
    )
    return deg, agg


_TC_BLK = 1000


def _hw_body(h_ref, w_ref, od_ref, o_ref):
    deg = od_ref[0] + od_ref[1]
    norm = lax.rsqrt(jnp.clip(deg[:, 0:1], 1.0, None))
    o_ref[...] = jnp.dot(h_ref[...] * norm, w_ref[...],
                         preferred_element_type=jnp.float32)


def _final_body(p_ref, id_ref, b_ref, o_ref):
    agg = p_ref[0] + p_ref[1]
    deg = id_ref[0] + id_ref[1]
    norm = lax.rsqrt(jnp.clip(deg[:, 0:1], 1.0, None))
    x = agg * norm + b_ref[...]
    o_ref[...] = jnp.where(x >= 0, x, 0.01 * x)


_hw_call = pl.pallas_call(
    _hw_body,
    grid=(N // _TC_BLK,),
    in_specs=[
        pl.BlockSpec((_TC_BLK, D), lambda i: (i, 0)),
        pl.BlockSpec((D, D), lambda i: (0, 0)),
        pl.BlockSpec((NC, _TC_BLK, 16), lambda i: (0, i, 0)),
    ],
    out_specs=pl.BlockSpec((_TC_BLK, D), lambda i: (i, 0)),
    out_shape=jax.ShapeDtypeStruct((N, D), jnp.float32),
)

_final_call = pl.pallas_call(
    _final_body,
    grid=(N // _TC_BLK,),
    in_specs=[
        pl.BlockSpec((NC, _TC_BLK, D), lambda i: (0, i, 0)),
        pl.BlockSpec((NC, _TC_BLK, 16), lambda i: (0, i, 0)),
        pl.BlockSpec((1, D), lambda i: (0, 0)),
    ],
    out_specs=pl.BlockSpec((_TC_BLK, D), lambda i: (i, 0)),
    out_shape=jax.ShapeDtypeStruct((N, D), jnp.float32),
)


def kernel(h, edge_index, W, b):
    idx2 = jnp.stack([edge_index[0].reshape(NW, CHUNKS, C),
                      edge_index[1].reshape(NW, CHUNKS, C)],
                     axis=2).reshape(NW, 2 * CHUNKS, C)
    ones16 = jnp.ones((C, 16), jnp.float32)
    zeros16 = jnp.zeros((NP, 16), jnp.float32)
    zerosD = jnp.zeros((NP, D), jnp.float32)

    deg_kernel, agg_kernel = _sc_kernels()
    odeg_p, ideg_p = deg_kernel(idx2, ones16, zeros16)
    hw = _hw_call(h, W, odeg_p)
    partials = agg_kernel(hw, idx2, zerosD)
    return _final_call(partials, ideg_p, b.reshape(1, D))


# speedup vs baseline: 1.1830x; 1.0659x over previous
import functools

import jax
import jax.numpy as jnp
from jax import lax
from jax.experimental import pallas as pl
from jax.experimental.pallas import tpu as pltpu
from jax.experimental.pallas import tpu_sc as plsc

N = 10000
E = 320000
D = 128

NC = 2
NS = 16
NW = NC * NS

EPT = E // NW
C = 125
CHUNKS = EPT // C
NP = 10240
RPT = NP // NS

def _deg_body(idx_hbm, ones_hbm, zeros_hbm,
                odeg_hbm, ideg_hbm,
                idx_v, ones_v, odeg_s, ideg_s, psem, ssem):
    cid = lax.axis_index("c")
    sid = lax.axis_index("s")
    wid = cid * NS + sid

    cps = [
        pltpu.async_copy(idx_hbm.at[0, wid], idx_v.at[0], psem),
        pltpu.async_copy(idx_hbm.at[1, wid], idx_v.at[1], psem),
        pltpu.async_copy(ones_hbm, ones_v, psem),
        pltpu.async_copy(zeros_hbm.at[pl.ds(sid * RPT, RPT)],
                         odeg_s.at[pl.ds(sid * RPT, RPT)], psem),
        pltpu.async_copy(zeros_hbm.at[pl.ds(sid * RPT, RPT)],
                         ideg_s.at[pl.ds(sid * RPT, RPT)], psem),
    ]
    for cp in cps:
        cp.wait()
    plsc.subcore_barrier()

    @pl.loop(0, CHUNKS, step=4)
    def step(p):
        fired = []
        for q in range(4):
            fired.append(pltpu.async_copy(
                ones_v, odeg_s.at[idx_v.at[0, p + q]], ssem, add=True))
            fired.append(pltpu.async_copy(
                ones_v, ideg_s.at[idx_v.at[1, p + q]], ssem, add=True))
        for cp in fired:
            cp.wait()

    plsc.subcore_barrier()

    pltpu.sync_copy(odeg_s.at[pl.ds(sid * RPT, RPT)],
                    odeg_hbm.at[cid, pl.ds(sid * RPT, RPT)])
    pltpu.sync_copy(ideg_s.at[pl.ds(sid * RPT, RPT)],
                    ideg_hbm.at[cid, pl.ds(sid * RPT, RPT)])


def _agg_body(hw_hbm, idx_hbm, zeros_hbm, out_hbm,
                sidx_a, sidx_b, didx_a, didx_b, rows_a, rows_b, acc_s,
                psem, isem_a, isem_b, gsem_a, gsem_b):
    cid = lax.axis_index("c")
    sid = lax.axis_index("s")
    wid = cid * NS + sid

    def stage(chunk, sbuf, dbuf, sem):
        pltpu.async_copy(idx_hbm.at[0, wid, chunk], sbuf, sem)
        pltpu.async_copy(idx_hbm.at[1, wid, chunk], dbuf, sem)

    def stage_wait(chunk, sbuf, dbuf, sem):
        pltpu.make_async_copy(idx_hbm.at[0, wid, chunk], sbuf, sem).wait()
        pltpu.make_async_copy(idx_hbm.at[1, wid, chunk], dbuf, sem).wait()

    zcp = pltpu.async_copy(zeros_hbm.at[pl.ds(sid * RPT, RPT)],
                           acc_s.at[pl.ds(sid * RPT, RPT)], psem)
    stage(0, sidx_a, didx_a, isem_a)
    stage_wait(0, sidx_a, didx_a, isem_a)
    pltpu.async_copy(hw_hbm.at[sidx_a], rows_a, gsem_a)
    stage(1, sidx_b, didx_b, isem_b)
    zcp.wait()
    plsc.subcore_barrier()

    @pl.loop(0, CHUNKS - 2, step=2)
    def pair(p):
        pltpu.make_async_copy(hw_hbm.at[sidx_a], rows_a, gsem_a).wait()
        stage_wait(p + 1, sidx_b, didx_b, isem_b)
        pltpu.async_copy(hw_hbm.at[sidx_b], rows_b, gsem_b)
        pltpu.sync_copy(rows_a, acc_s.at[didx_a], add=True)
        stage(p + 2, sidx_a, didx_a, isem_a)
        pltpu.make_async_copy(hw_hbm.at[sidx_b], rows_b, gsem_b).wait()
        stage_wait(p + 2, sidx_a, didx_a, isem_a)
        pltpu.async_copy(hw_hbm.at[sidx_a], rows_a, gsem_a)
        pltpu.sync_copy(rows_b, acc_s.at[didx_b], add=True)
        stage(p + 3, sidx_b, didx_b, isem_b)

    p = CHUNKS - 2
    pltpu.make_async_copy(hw_hbm.at[sidx_a], rows_a, gsem_a).wait()
    stage_wait(p + 1, sidx_b, didx_b, isem_b)
    pltpu.async_copy(hw_hbm.at[sidx_b], rows_b, gsem_b)
    pltpu.sync_copy(rows_a, acc_s.at[didx_a], add=True)
    pltpu.make_async_copy(hw_hbm.at[sidx_b], rows_b, gsem_b).wait()
    pltpu.sync_copy(rows_b, acc_s.at[didx_b], add=True)

    plsc.subcore_barrier()

    pltpu.sync_copy(acc_s.at[pl.ds(sid * RPT, RPT)],
                    out_hbm.at[cid, pl.ds(sid * RPT, RPT)])


@functools.cache
def _sc_kernels():
    mesh = plsc.VectorSubcoreMesh(core_axis_name="c", subcore_axis_name="s",
                                  num_cores=NC, num_subcores=NS)
    deg = pl.kernel(
        _deg_body,
        out_type=[
            jax.ShapeDtypeStruct((NC, NP, 16), jnp.float32),
            jax.ShapeDtypeStruct((NC, NP, 16), jnp.float32),
        ],
        mesh=mesh,
        scratch_types=[
            pltpu.VMEM((2, CHUNKS, C), jnp.int32),
            pltpu.VMEM((C, 16), jnp.float32),
            pltpu.VMEM_SHARED((NP, 16), jnp.float32),
            pltpu.VMEM_SHARED((NP, 16), jnp.float32),
            pltpu.SemaphoreType.DMA,
            pltpu.SemaphoreType.DMA,
        ],
        compiler_params=pltpu.CompilerParams(use_tc_tiling_on_sc=False),
    )
    agg = pl.kernel(
        _agg_body,
        out_type=jax.ShapeDtypeStruct((NC, NP, D), jnp.float32),
        mesh=mesh,
        scratch_types=[
            pltpu.VMEM((C,), jnp.int32),
            pltpu.VMEM((C,), jnp.int32),
            pltpu.VMEM((C,), jnp.int32),
            pltpu.VMEM((C,), jnp.int32),
            pltpu.VMEM((C, D), jnp.float32),
            pltpu.VMEM((C, D), jnp.float32),
            pltpu.VMEM_SHARED((NP, D), jnp.float32),
            pltpu.SemaphoreType.DMA,
            pltpu.SemaphoreType.DMA,
            pltpu.SemaphoreType.DMA,
            pltpu.SemaphoreType.DMA,
            pltpu.SemaphoreType.DMA,
        ],
        compiler_params=pltpu.CompilerParams(use_tc_tiling_on_sc=False),
    )
    return deg, agg


_TC_BLK = 1000


def _hw_body(h_ref, w_ref, od_ref, o_ref):
    deg = od_ref[0] + od_ref[1]
    norm = lax.rsqrt(jnp.clip(deg[:, 0:1], 1.0, None))
    o_ref[...] = jnp.dot(h_ref[...] * norm, w_ref[...],
                         preferred_element_type=jnp.float32)


def _final_body(p_ref, id_ref, b_ref, o_ref):
    agg = p_ref[0] + p_ref[1]
    deg = id_ref[0] + id_ref[1]
    norm = lax.rsqrt(jnp.clip(deg[:, 0:1], 1.0, None))
    x = agg * norm + b_ref[...]
    o_ref[...] = jnp.where(x >= 0, x, 0.01 * x)


_hw_call = pl.pallas_call(
    _hw_body,
    grid=(N // _TC_BLK,),
    in_specs=[
        pl.BlockSpec((_TC_BLK, D), lambda i: (i, 0)),
        pl.BlockSpec((D, D), lambda i: (0, 0)),
        pl.BlockSpec((NC, _TC_BLK, 16), lambda i: (0, i, 0)),
    ],
    out_specs=pl.BlockSpec((_TC_BLK, D), lambda i: (i, 0)),
    out_shape=jax.ShapeDtypeStruct((N, D), jnp.float32),
)

_final_call = pl.pallas_call(
    _final_body,
    grid=(N // _TC_BLK,),
    in_specs=[
        pl.BlockSpec((NC, _TC_BLK, D), lambda i: (0, i, 0)),
        pl.BlockSpec((NC, _TC_BLK, 16), lambda i: (0, i, 0)),
        pl.BlockSpec((1, D), lambda i: (0, 0)),
    ],
    out_specs=pl.BlockSpec((_TC_BLK, D), lambda i: (i, 0)),
    out_shape=jax.ShapeDtypeStruct((N, D), jnp.float32),
)


def kernel(h, edge_index, W, b):
    eidx = edge_index.reshape(2, NW, CHUNKS, C)
    ones16 = jnp.ones((C, 16), jnp.float32)
    zeros16 = jnp.zeros((NP, 16), jnp.float32)
    zerosD = jnp.zeros((NP, D), jnp.float32)

    deg_kernel, agg_kernel = _sc_kernels()
    odeg_p, ideg_p = deg_kernel(eidx, ones16, zeros16)
    hw = _hw_call(h, W, odeg_p)
    partials = agg_kernel(hw, eidx, zerosD)
    return _final_call(partials, ideg_p, b.reshape(1, D))
